# baseline (speedup 1.0000x reference)
import functools

import jax
import jax.numpy as jnp
from jax import lax
from jax.experimental import pallas as pl
from jax.experimental.pallas import tpu as pltpu
from jax.experimental.pallas import tpu_sc as plsc

VOCAB = 1000000
DIM = 128
RANK = 32
BATCH = 4096
SEQ = 200
N_TOK = BATCH * SEQ
N4 = N_TOK // 4

_info = plsc.get_sparse_core_info()
NC, NS = _info.num_cores, _info.num_subcores
NW = NC * NS
TOK_PER_W = N_TOK // NW
CHUNK_T = 1600
CHUNK_L = CHUNK_T // 4
N_CHUNKS = TOK_PER_W // CHUNK_T

_sc_mesh = plsc.VectorSubcoreMesh(core_axis_name="c", subcore_axis_name="s")


@functools.partial(
    pl.kernel,
    mesh=_sc_mesh,
    out_type=jax.ShapeDtypeStruct((N4, 128), jnp.float32),
    scratch_types=[
        pltpu.VMEM((CHUNK_T,), jnp.int32),
        pltpu.VMEM((CHUNK_T, RANK), jnp.float32),
        pltpu.SemaphoreType.DMA,
    ],
    compiler_params=pltpu.CompilerParams(use_tc_tiling_on_sc=False),
)
def _sc_gather(table_hbm, idx_hbm, out_hbm, idx_v, rows_v, sem):
    wid = lax.axis_index("s") * NC + lax.axis_index("c")
    tbase = wid * TOK_PER_W
    lbase = wid * (TOK_PER_W // 4)

    def chunk_body(c, carry):
        toff = tbase + c * CHUNK_T
        loff = lbase + c * CHUNK_L
        pltpu.sync_copy(idx_hbm.at[pl.ds(toff, CHUNK_T)], idx_v)
        copies = [
            pltpu.async_copy(
                table_hbm.at[idx_v.at[pl.ds(CHUNK_L * a, CHUNK_L)]],
                rows_v.at[pl.ds(CHUNK_L * a, CHUNK_L)],
                sem,
            )
            for a in range(4)
        ]
        for cp in copies:
            cp.wait()
        for a in range(4):
            pltpu.sync_copy(
                rows_v.at[pl.ds(CHUNK_L * a, CHUNK_L)],
                out_hbm.at[pl.ds(loff, CHUNK_L), pl.ds(RANK * a, RANK)],
            )
        return carry

    lax.fori_loop(0, N_CHUNKS, chunk_body, 0)


CH_PER_STEP = 16
STEP_L = CH_PER_STEP * CHUNK_L
STEP_T = CH_PER_STEP * CHUNK_T
STEP_B = STEP_T // SEQ


def _mm_body(u4_ref, w_ref, o_ref):
    w = w_ref[...]
    outs = []
    for g in range(CH_PER_STEP):
        u = u4_ref[pl.ds(CHUNK_L * g, CHUNK_L), :].astype(jnp.bfloat16)
        p = jnp.dot(u, w, preferred_element_type=jnp.float32)
        outs.extend(p[:, DIM * a:DIM * (a + 1)] for a in range(4))
    o_ref[...] = jnp.concatenate(outs, axis=0).reshape(STEP_B, SEQ, DIM)


def kernel(token_ids, U, V):
    ids = token_ids.reshape(-1).astype(jnp.int32)
    u4 = _sc_gather(U, ids)
    w = jnp.kron(jnp.eye(4, dtype=jnp.float32), V).astype(jnp.bfloat16)

    out = pl.pallas_call(
        _mm_body,
        grid=(N_TOK // STEP_T,),
        in_specs=[
            pl.BlockSpec((STEP_L, 128), lambda i: (i, 0)),
            pl.BlockSpec((128, 4 * DIM), lambda i: (0, 0)),
        ],
        out_specs=pl.BlockSpec((STEP_B, SEQ, DIM), lambda i: (i, 0, 0)),
        out_shape=jax.ShapeDtypeStruct((BATCH, SEQ, DIM), jnp.float32),
    )(u4, w)
    return out


# speedup vs baseline: 1.0073x; 1.0073x over previous
import functools

import jax
import jax.numpy as jnp
from jax import lax
from jax.experimental import pallas as pl
from jax.experimental.pallas import tpu as pltpu
from jax.experimental.pallas import tpu_sc as plsc

VOCAB = 1000000
DIM = 128
RANK = 32
BATCH = 4096
SEQ = 200
N_TOK = BATCH * SEQ

P = 4
N_SL = N_TOK // P
L_SL = N_SL // 4

_info = plsc.get_sparse_core_info()
NC, NS = _info.num_cores, _info.num_subcores
NW = NC * NS
TOK_PER_W = N_SL // NW
CHUNK_T = 1600
CHUNK_L = CHUNK_T // 4
N_CHUNKS = TOK_PER_W // CHUNK_T

_sc_mesh = plsc.VectorSubcoreMesh(core_axis_name="c", subcore_axis_name="s")


@functools.partial(
    pl.kernel,
    mesh=_sc_mesh,
    out_type=jax.ShapeDtypeStruct((L_SL, 128), jnp.float32),
    scratch_types=[
        pltpu.VMEM((CHUNK_T,), jnp.int32),
        pltpu.VMEM((CHUNK_T, RANK), jnp.float32),
        pltpu.SemaphoreType.DMA,
    ],
    compiler_params=pltpu.CompilerParams(use_tc_tiling_on_sc=False),
)
def _sc_gather(table_hbm, idx_hbm, out_hbm, idx_v, rows_v, sem):
    wid = lax.axis_index("s") * NC + lax.axis_index("c")
    tbase = wid * TOK_PER_W
    lbase = wid * (TOK_PER_W // 4)

    def chunk_body(c, carry):
        toff = tbase + c * CHUNK_T
        loff = lbase + c * CHUNK_L
        pltpu.sync_copy(idx_hbm.at[pl.ds(toff, CHUNK_T)], idx_v)
        copies = [
            pltpu.async_copy(
                table_hbm.at[idx_v.at[pl.ds(CHUNK_L * a, CHUNK_L)]],
                rows_v.at[pl.ds(CHUNK_L * a, CHUNK_L)],
                sem,
            )
            for a in range(4)
        ]
        for cp in copies:
            cp.wait()
        for a in range(4):
            pltpu.sync_copy(
                rows_v.at[pl.ds(CHUNK_L * a, CHUNK_L)],
                out_hbm.at[pl.ds(loff, CHUNK_L), pl.ds(RANK * a, RANK)],
            )
        return carry

    lax.fori_loop(0, N_CHUNKS, chunk_body, 0)


CH_PER_STEP = 16
STEP_L = CH_PER_STEP * CHUNK_L
STEP_T = CH_PER_STEP * CHUNK_T
STEP_B = STEP_T // SEQ
STEPS_PER_SLICE = N_SL // STEP_T


def _mm_compute(u4_ref, w_ref, o_ref):
    w = w_ref[...]
    outs = []
    for g in range(CH_PER_STEP):
        u = u4_ref[pl.ds(CHUNK_L * g, CHUNK_L), :].astype(jnp.bfloat16)
        p = jnp.dot(u, w, preferred_element_type=jnp.float32)
        outs.extend(p[:, DIM * a:DIM * (a + 1)] for a in range(4))
    o_ref[...] = jnp.concatenate(outs, axis=0).reshape(STEP_B, SEQ, DIM)


def _mm_first(u4_ref, w_ref, o_ref):
    _mm_compute(u4_ref, w_ref, o_ref)


def _mm_next(u4_ref, w_ref, prev_ref, o_ref):
    del prev_ref
    _mm_compute(u4_ref, w_ref, o_ref)


def kernel(token_ids, U, V):
    ids = token_ids.reshape(-1).astype(jnp.int32)
    w = jnp.kron(jnp.eye(4, dtype=jnp.float32), V).astype(jnp.bfloat16)
    out_shape = jax.ShapeDtypeStruct((BATCH, SEQ, DIM), jnp.float32)

    u4s = [
        _sc_gather(U, lax.slice_in_dim(ids, p * N_SL, (p + 1) * N_SL))
        for p in range(P)
    ]

    def out_spec(p):
        return pl.BlockSpec(
            (STEP_B, SEQ, DIM),
            lambda i, p=p: (STEPS_PER_SLICE * p + i, 0, 0),
        )

    in_specs = [
        pl.BlockSpec((STEP_L, 128), lambda i: (i, 0)),
        pl.BlockSpec((128, 4 * DIM), lambda i: (0, 0)),
    ]

    out = pl.pallas_call(
        _mm_first,
        grid=(STEPS_PER_SLICE,),
        in_specs=in_specs,
        out_specs=out_spec(0),
        out_shape=out_shape,
    )(u4s[0], w)

    for p in range(1, P):
        out = pl.pallas_call(
            _mm_next,
            grid=(STEPS_PER_SLICE,),
            in_specs=in_specs + [pl.BlockSpec(memory_space=pl.ANY)],
            out_specs=out_spec(p),
            out_shape=out_shape,
            input_output_aliases={2: 0},
        )(u4s[p], w, out)

    return out
